# baseline (speedup 1.0000x reference)
import functools

import jax
import jax.numpy as jnp
from jax import lax
from jax.experimental import pallas as pl
from jax.experimental.pallas import tpu as pltpu
from jax.experimental.pallas import tpu_sc as plsc

N_CLS = 1000
N_CTX = 16
CTX_DIM = 512
N_PROMPTS = 32
SEQ = 77
B = 4096
SUFFIX_LEN = SEQ - 1 - N_CTX

_info = plsc.get_sparse_core_info()
_NC = _info.num_cores
_NS = _info.num_subcores
_NW = _NC * _NS
_BPW = B // _NW
_GRP = 16
_NGRP = _BPW // _GRP

_mesh = plsc.VectorSubcoreMesh(core_axis_name="c", subcore_axis_name="s")


@functools.partial(
    pl.kernel,
    mesh=_mesh,
    out_type=jax.ShapeDtypeStruct((B, SEQ, CTX_DIM), jnp.float32),
    scratch_types=[
        pltpu.VMEM((_BPW,), jnp.int32),
        pltpu.VMEM((_BPW,), jnp.int32),
        pltpu.SemaphoreType.DMA,
    ],
    compiler_params=pltpu.CompilerParams(use_tc_tiling_on_sc=False,
                                         needs_layout_passes=False),
)
def _sc_concat_gather(ctx_hbm, prefix_hbm, suffix_hbm, labels_hbm, match_hbm,
                      out_hbm, labels_v, match_v, sem):
    wid = lax.axis_index("s") * _NC + lax.axis_index("c")
    base = wid * _BPW
    pltpu.sync_copy(labels_hbm.at[pl.ds(base, _BPW)], labels_v)
    pltpu.sync_copy(match_hbm.at[pl.ds(base, _BPW)], match_v)
    lane_ids = lax.iota(jnp.int32, 16)

    def group(g):
        start = pl.multiple_of(g * _GRP, _GRP)
        lvec = labels_v[pl.ds(start, _GRP)]
        mvec = match_v[pl.ds(start, _GRP)]
        for lane in range(_GRP):
            lbl = jnp.sum(jnp.where(lane_ids == lane, lvec, 0))
            mid = jnp.sum(jnp.where(lane_ids == lane, mvec, 0))
            row = out_hbm.at[pl.ds(base + g * _GRP + lane, 1)]
            pltpu.async_copy(prefix_hbm.at[pl.ds(lbl, 1)],
                             row.at[:, pl.ds(0, 1)], sem)
            pltpu.async_copy(ctx_hbm.at[pl.ds(mid, 1)],
                             row.at[:, pl.ds(1, N_CTX)], sem)
            pltpu.async_copy(suffix_hbm.at[pl.ds(lbl, 1)],
                             row.at[:, pl.ds(1 + N_CTX, SUFFIX_LEN)], sem)

    pl.loop(0, _NGRP)(group)

    chunk = out_hbm.at[pl.ds(base, _BPW)]
    pltpu.make_async_copy(chunk, chunk, sem).wait()


def kernel(ctx, token_prefix, token_suffix, labels, match_ids):
    return _sc_concat_gather(ctx, token_prefix, token_suffix,
                             labels.astype(jnp.int32),
                             match_ids.astype(jnp.int32))
